# baseline (speedup 1.0000x reference)
import functools

import jax
import jax.numpy as jnp
from jax import lax
from jax.experimental import pallas as pl
from jax.experimental.pallas import tpu as pltpu
from jax.experimental.pallas import tpu_sc as plsc

N = 5000
NP = 5120
W = 160
NB = 32
NS = 16
RT = NP // NS
L = 16
THR = 0.5
NEG_INF = float("-inf")


def _scores_classes_cols(x4):
    s = jnp.max(x4, axis=1, keepdims=True)
    c0 = x4[:, 0:1] == s
    c1 = x4[:, 1:2] == s
    c2 = x4[:, 2:3] == s
    cls = jnp.where(c0, 0, jnp.where(c1, 1, jnp.where(c2, 2, 3)))
    return s, cls.astype(jnp.int32)


def _build_body(xp_ref, rp_ref, xj_ref, bj_ref, at_ref, scols_ref):
    i32 = jnp.int32
    g = pl.program_id(0)

    @pl.when(g == 0)
    def _prep():
        x4 = xp_ref[:, :]
        s, cls = _scores_classes_cols(x4)
        seff = jnp.where(cls != 3, s, NEG_INF)
        r4 = rp_ref[:, :]
        area = (jnp.maximum(r4[:, 2:3] - r4[:, 0:1], 0.0)
                * jnp.maximum(r4[:, 3:4] - r4[:, 1:2], 0.0))
        scols_ref[:, :] = jnp.concatenate(
            [r4, area, seff, seff, seff], axis=1)

    xjb = xj_ref[0]
    sj = jnp.max(xjb[0:4, :], axis=0, keepdims=True)
    inv_j = ((xjb[3:4, :] > xjb[0:1, :])
             & (xjb[3:4, :] > xjb[1:2, :])
             & (xjb[3:4, :] > xjb[2:3, :]))
    sj_eff = jnp.where(inv_j, NEG_INF, sj)
    bjb = bj_ref[0]
    x1j, y1j = bjb[0:1, :], bjb[1:2, :]
    x2j, y2j = bjb[2:3, :], bjb[3:4, :]
    area_j = jnp.maximum(x2j - x1j, 0.0) * jnp.maximum(y2j - y1j, 0.0)
    idx_j = lax.broadcasted_iota(i32, (1, RT), 1) + g * RT

    acc = jnp.zeros((W, RT), i32)
    for b in range(NB):
        sc = scols_ref[W * b:W * (b + 1), :]
        x1i, y1i = sc[:, 0:1], sc[:, 1:2]
        x2i, y2i = sc[:, 2:3], sc[:, 3:4]
        area_i = sc[:, 4:5]
        si_eff = sc[:, 5:6]
        idx_i = lax.broadcasted_iota(i32, (W, 1), 0) + W * b

        iw = jnp.maximum(jnp.minimum(x2j, x2i) - jnp.maximum(x1j, x1i), 0.0)
        ih = jnp.maximum(jnp.minimum(y2j, y2i) - jnp.maximum(y1j, y1i), 0.0)
        inter = iw * ih
        union = jnp.maximum(area_i + area_j - inter, 1e-9)
        conflict = inter > THR * union
        prec = (si_eff > sj_eff) | ((si_eff == sj_eff) & (idx_i < idx_j))
        acc = acc | lax.shift_left((conflict & prec).astype(i32), b)
    at_ref[0] = acc


NR = 12


def _solve_body(at3, keep_in, xt_f, rt_f, keep_out, diff_out,
                osc, olab, obox_f,
                at_v, keep_v, nk_v, kw_v, ko_v, shk,
                xc0, xc1, xc2, xc3, rc0, rc1, rc2, rc3,
                osc_v, olab_v, ob0, ob1, ob2, ob3):
    f32 = jnp.float32
    i32 = jnp.int32
    cid = lax.axis_index("c")
    sid = lax.axis_index("s")
    tid = sid
    base = pl.multiple_of(tid * RT, RT)
    pltpu.sync_copy(at3.at[tid], at_v)
    pltpu.sync_copy(keep_in, keep_v)

    nch = RT // L

    def one_round(src, dst, slot):
        diff = jnp.zeros((L,), i32)
        for c in range(W // L):
            kwc = jnp.zeros((L,), i32)
            for b in range(NB):
                kwc = kwc | lax.shift_left(
                    src[pl.ds(W * b + L * c, L)], b)
            diff = diff | (kwc ^ kw_v[pl.ds(L * c, L)])
            kw_v[pl.ds(L * c, L)] = kwc

        def wcbody(wc, accs):
            woff = pl.multiple_of(L * wc, L)
            kwc = kw_v[pl.ds(woff, L)]
            for l in range(L):
                kb = jnp.full((L,), kwc[l], i32)
                w = woff + l
                accs = tuple(
                    accs[rc] | (at_v[w, pl.ds(L * rc, L)] & kb)
                    for rc in range(nch))
            return accs

        accs = lax.fori_loop(
            0, W // L, wcbody,
            tuple(jnp.zeros((L,), i32) for _ in range(nch)))
        for rc in range(nch):
            ko_v[pl.ds(L * rc, L)] = jnp.where(
                accs[rc] == 0, 1, 0).astype(i32)

        pltpu.sync_copy(ko_v, shk.at[pl.ds(slot * NP + base, RT)])
        plsc.subcore_barrier()
        pltpu.sync_copy(shk.at[pl.ds(slot * NP, NP)], dst)
        return diff

    def round_pair(r, _):
        one_round(keep_v, nk_v, 0)
        return one_round(nk_v, keep_v, 1)

    zl = jnp.zeros((L,), i32)
    for c in range(W // L):
        kw_v[pl.ds(L * c, L)] = zl
    lax.fori_loop(0, NR // 2, round_pair, zl)
    diff = zl
    for c in range(W // L):
        kwc = jnp.zeros((L,), i32)
        for b in range(NB):
            kwc = kwc | lax.shift_left(
                keep_v[pl.ds(W * b + L * c, L)], b)
        diff = diff | (kwc ^ kw_v[pl.ds(L * c, L)])

    obase = pl.multiple_of(cid * NP, NP) + base
    for rc in range(nch):
        ko_v[pl.ds(L * rc, L)] = keep_v[pl.ds(base + L * rc, L)]
    pltpu.sync_copy(ko_v, keep_out.at[pl.ds(obase, RT)])
    for rc in range(nch):
        ko_v[pl.ds(L * rc, L)] = diff
    pltpu.sync_copy(ko_v, diff_out.at[pl.ds(obase, RT)])

    xcs = (xc0, xc1, xc2, xc3)
    rcs = (rc0, rc1, rc2, rc3)
    obs = (ob0, ob1, ob2, ob3)
    for c in range(4):
        pltpu.sync_copy(xt_f.at[pl.ds(c * NP + base, RT)], xcs[c])
        pltpu.sync_copy(rt_f.at[pl.ds(c * NP + base, RT)], rcs[c])
    for k in range(nch):
        sl = pl.ds(L * k, L)
        x0, x1, x2, x3 = xc0[sl], xc1[sl], xc2[sl], xc3[sl]
        s = jnp.maximum(jnp.maximum(x0, x1), jnp.maximum(x2, x3))
        is3 = (x3 > x0) & (x3 > x1) & (x3 > x2)
        cls = jnp.where(x0 == s, 0,
                        jnp.where(x1 == s, 1,
                                  jnp.where(x2 == s, 2, 3))).astype(i32)
        kv = keep_v[pl.ds(base + L * k, L)] * jnp.where(is3, 0, 1)
        kf = kv.astype(f32)
        osc_v[sl] = s * kf
        olab_v[sl] = cls * kv
        for c in range(4):
            obs[c][sl] = rcs[c][sl] * kf
    pltpu.sync_copy(osc_v, osc.at[pl.ds(obase, RT)])
    pltpu.sync_copy(olab_v, olab.at[pl.ds(obase, RT)])
    ob_base = pl.multiple_of(cid * 4 * NP, NP) + base
    for c in range(4):
        pltpu.sync_copy(obs[c], obox_f.at[pl.ds(ob_base + c * NP, RT)])


_SC_MESH = plsc.VectorSubcoreMesh(core_axis_name="c", subcore_axis_name="s")

_sc_solve = functools.partial(
    pl.kernel,
    out_type=[
        jax.ShapeDtypeStruct((2 * NP,), jnp.int32),
        jax.ShapeDtypeStruct((2 * NP,), jnp.int32),
        jax.ShapeDtypeStruct((2 * NP,), jnp.float32),
        jax.ShapeDtypeStruct((2 * NP,), jnp.int32),
        jax.ShapeDtypeStruct((8 * NP,), jnp.float32),
    ],
    mesh=_SC_MESH,
    scratch_types=(
        [pltpu.VMEM((W, RT), jnp.int32),
         pltpu.VMEM((NP,), jnp.int32),
         pltpu.VMEM((NP,), jnp.int32),
         pltpu.VMEM((W,), jnp.int32),
         pltpu.VMEM((RT,), jnp.int32),
         pltpu.VMEM_SHARED((2 * NP,), jnp.int32)]
        + [pltpu.VMEM((RT,), jnp.float32)] * 8
        + [pltpu.VMEM((RT,), jnp.float32),
           pltpu.VMEM((RT,), jnp.int32)]
        + [pltpu.VMEM((RT,), jnp.float32)] * 4
    ),
)(_solve_body)


@jax.jit
def kernel(x, rois):
    f32 = jnp.float32
    i32 = jnp.int32
    xp = jnp.zeros((NP, 4), f32).at[:N, :].set(x)
    rp = jnp.zeros((NP, 4), f32).at[:N, :].set(rois)
    xj3 = jnp.zeros((NS, 8, RT), f32).at[:, :4, :].set(
        xp.reshape(NS, RT, 4).transpose(0, 2, 1))
    bj3 = jnp.zeros((NS, 8, RT), f32).at[:, :4, :].set(
        rp.reshape(NS, RT, 4).transpose(0, 2, 1))

    at3 = pl.pallas_call(
        _build_body,
        grid=(NS,),
        in_specs=[
            pl.BlockSpec((NP, 4), lambda g: (0, 0)),
            pl.BlockSpec((NP, 4), lambda g: (0, 0)),
            pl.BlockSpec((1, 8, RT), lambda g: (g, 0, 0)),
            pl.BlockSpec((1, 8, RT), lambda g: (g, 0, 0)),
        ],
        out_specs=pl.BlockSpec((1, W, RT), lambda g: (g, 0, 0)),
        out_shape=jax.ShapeDtypeStruct((NS, W, RT), i32),
        scratch_shapes=[pltpu.VMEM((NP, 8), f32)],
    )(xp, rp, xj3, bj3)

    xt_f = xp.T.reshape(-1)
    rt_f = rp.T.reshape(-1)

    def cond(st):
        return st[1]

    def body(st):
        keep = st[0]
        keep2, diff2, osc2, olab2, obf2 = _sc_solve(at3, keep, xt_f, rt_f)
        return (keep2[:NP], jnp.any(diff2[:NP] != 0),
                osc2[:NP], olab2[:NP], obf2[:4 * NP])

    st0 = (jnp.ones((NP,), i32), jnp.bool_(True),
           jnp.zeros((NP,), f32), jnp.zeros((NP,), i32),
           jnp.zeros((4 * NP,), f32))
    _, _, osc, olab, obox_f = lax.while_loop(cond, body, st0)
    obox = obox_f.reshape(4, NP).T
    return osc[:N], olab[:N], obox[:N, :]


# speedup vs baseline: 1.0806x; 1.0806x over previous
import functools

import jax
import jax.numpy as jnp
from jax import lax
from jax.experimental import pallas as pl
from jax.experimental.pallas import tpu as pltpu
from jax.experimental.pallas import tpu_sc as plsc

N = 5000
NP = 5120
W = 160
NB = 32
NS = 16
RT = NP // NS
L = 16
THR = 0.5
NEG_INF = float("-inf")


def _scores_classes_cols(x4):
    s = jnp.max(x4, axis=1, keepdims=True)
    c0 = x4[:, 0:1] == s
    c1 = x4[:, 1:2] == s
    c2 = x4[:, 2:3] == s
    cls = jnp.where(c0, 0, jnp.where(c1, 1, jnp.where(c2, 2, 3)))
    return s, cls.astype(jnp.int32)


def _build_body(xp_ref, rp_ref, xj_ref, bj_ref, at_ref, scols_ref):
    i32 = jnp.int32
    g = pl.program_id(0)

    @pl.when(g == 0)
    def _prep():
        x4 = xp_ref[:, :]
        s, cls = _scores_classes_cols(x4)
        seff = jnp.where(cls != 3, s, NEG_INF)
        r4 = rp_ref[:, :]
        area = (jnp.maximum(r4[:, 2:3] - r4[:, 0:1], 0.0)
                * jnp.maximum(r4[:, 3:4] - r4[:, 1:2], 0.0))
        scols_ref[:, :] = jnp.concatenate(
            [r4, area, seff, seff, seff], axis=1)

    xjb = xj_ref[0]
    sj = jnp.max(xjb[0:4, :], axis=0, keepdims=True)
    inv_j = ((xjb[3:4, :] > xjb[0:1, :])
             & (xjb[3:4, :] > xjb[1:2, :])
             & (xjb[3:4, :] > xjb[2:3, :]))
    sj_eff = jnp.where(inv_j, NEG_INF, sj)
    bjb = bj_ref[0]
    x1j, y1j = bjb[0:1, :], bjb[1:2, :]
    x2j, y2j = bjb[2:3, :], bjb[3:4, :]
    area_j = jnp.maximum(x2j - x1j, 0.0) * jnp.maximum(y2j - y1j, 0.0)
    idx_j = lax.broadcasted_iota(i32, (1, RT), 1) + g * RT

    acc = jnp.zeros((W, RT), i32)
    for b in range(NB):
        sc = scols_ref[W * b:W * (b + 1), :]
        x1i, y1i = sc[:, 0:1], sc[:, 1:2]
        x2i, y2i = sc[:, 2:3], sc[:, 3:4]
        area_i = sc[:, 4:5]
        si_eff = sc[:, 5:6]
        idx_i = lax.broadcasted_iota(i32, (W, 1), 0) + W * b

        iw = jnp.maximum(jnp.minimum(x2j, x2i) - jnp.maximum(x1j, x1i), 0.0)
        ih = jnp.maximum(jnp.minimum(y2j, y2i) - jnp.maximum(y1j, y1i), 0.0)
        inter = iw * ih
        union = jnp.maximum(area_i + area_j - inter, 1e-9)
        conflict = inter > THR * union
        prec = (si_eff > sj_eff) | ((si_eff == sj_eff) & (idx_i < idx_j))
        acc = acc | lax.shift_left((conflict & prec).astype(i32), b)
    at_ref[0] = acc


NR = 12


def _solve_body(at3, keep_in, keep_out, diff_out,
                at_v, keep_v, nk_v, kw_v, ko_v, shk):
    i32 = jnp.int32
    cid = lax.axis_index("c")
    sid = lax.axis_index("s")
    tid = sid
    base = pl.multiple_of(tid * RT, RT)
    pltpu.sync_copy(at3.at[tid], at_v)
    pltpu.sync_copy(keep_in, keep_v)

    nch = RT // L

    def one_round(src, dst, slot):
        diff = jnp.zeros((L,), i32)
        for c in range(W // L):
            kwc = jnp.zeros((L,), i32)
            for b in range(NB):
                kwc = kwc | lax.shift_left(
                    src[pl.ds(W * b + L * c, L)], b)
            diff = diff | (kwc ^ kw_v[pl.ds(L * c, L)])
            kw_v[pl.ds(L * c, L)] = kwc

        def wcbody(wc, accs):
            woff = pl.multiple_of(L * wc, L)
            kwc = kw_v[pl.ds(woff, L)]
            for l in range(L):
                kb = jnp.full((L,), kwc[l], i32)
                w = woff + l
                accs = tuple(
                    accs[rc] | (at_v[w, pl.ds(L * rc, L)] & kb)
                    for rc in range(nch))
            return accs

        accs = lax.fori_loop(
            0, W // L, wcbody,
            tuple(jnp.zeros((L,), i32) for _ in range(nch)))
        for rc in range(nch):
            ko_v[pl.ds(L * rc, L)] = jnp.where(
                accs[rc] == 0, 1, 0).astype(i32)

        pltpu.sync_copy(ko_v, shk.at[pl.ds(slot * NP + base, RT)])
        plsc.subcore_barrier()
        pltpu.sync_copy(shk.at[pl.ds(slot * NP, NP)], dst)
        return diff

    def round_pair(r, _):
        one_round(keep_v, nk_v, 0)
        return one_round(nk_v, keep_v, 1)

    zl = jnp.zeros((L,), i32)
    for c in range(W // L):
        kw_v[pl.ds(L * c, L)] = zl
    lax.fori_loop(0, NR // 2, round_pair, zl)
    diff = zl
    for c in range(W // L):
        kwc = jnp.zeros((L,), i32)
        for b in range(NB):
            kwc = kwc | lax.shift_left(
                keep_v[pl.ds(W * b + L * c, L)], b)
        diff = diff | (kwc ^ kw_v[pl.ds(L * c, L)])

    obase = pl.multiple_of(cid * NP, NP) + base
    for rc in range(nch):
        ko_v[pl.ds(L * rc, L)] = keep_v[pl.ds(base + L * rc, L)]
    pltpu.sync_copy(ko_v, keep_out.at[pl.ds(obase, RT)])
    for rc in range(nch):
        ko_v[pl.ds(L * rc, L)] = diff
    pltpu.sync_copy(ko_v, diff_out.at[pl.ds(obase, RT)])


_SC_MESH = plsc.VectorSubcoreMesh(core_axis_name="c", subcore_axis_name="s")

_sc_solve = functools.partial(
    pl.kernel,
    out_type=[
        jax.ShapeDtypeStruct((2 * NP,), jnp.int32),
        jax.ShapeDtypeStruct((2 * NP,), jnp.int32),
    ],
    mesh=_SC_MESH,
    scratch_types=[
        pltpu.VMEM((W, RT), jnp.int32),
        pltpu.VMEM((NP,), jnp.int32),
        pltpu.VMEM((NP,), jnp.int32),
        pltpu.VMEM((W,), jnp.int32),
        pltpu.VMEM((RT,), jnp.int32),
        pltpu.VMEM_SHARED((2 * NP,), jnp.int32),
    ],
)(_solve_body)


RTE = NP // 32


def _emit_body(xt_f, rt_f, keep_in, osc, olab, obox_f,
               xc0, xc1, xc2, xc3, rc0, rc1, rc2, rc3,
               keep_v, osc_v, olab_v, ob0, ob1, ob2, ob3):
    f32 = jnp.float32
    i32 = jnp.int32
    wid = lax.axis_index("s") * 2 + lax.axis_index("c")
    base = pl.multiple_of(wid * RTE, RTE)

    xcs = (xc0, xc1, xc2, xc3)
    rcs = (rc0, rc1, rc2, rc3)
    obs = (ob0, ob1, ob2, ob3)
    for c in range(4):
        pltpu.sync_copy(xt_f.at[pl.ds(c * NP + base, RTE)], xcs[c])
        pltpu.sync_copy(rt_f.at[pl.ds(c * NP + base, RTE)], rcs[c])
    pltpu.sync_copy(keep_in.at[pl.ds(base, RTE)], keep_v)

    for k in range(RTE // L):
        sl = pl.ds(L * k, L)
        x0, x1, x2, x3 = xc0[sl], xc1[sl], xc2[sl], xc3[sl]
        s = jnp.maximum(jnp.maximum(x0, x1), jnp.maximum(x2, x3))
        is3 = (x3 > x0) & (x3 > x1) & (x3 > x2)
        cls = jnp.where(x0 == s, 0,
                        jnp.where(x1 == s, 1,
                                  jnp.where(x2 == s, 2, 3))).astype(i32)
        kv = keep_v[sl] * jnp.where(is3, 0, 1)
        kf = kv.astype(f32)
        osc_v[sl] = s * kf
        olab_v[sl] = cls * kv
        for c in range(4):
            obs[c][sl] = rcs[c][sl] * kf
    pltpu.sync_copy(osc_v, osc.at[pl.ds(base, RTE)])
    pltpu.sync_copy(olab_v, olab.at[pl.ds(base, RTE)])
    for c in range(4):
        pltpu.sync_copy(obs[c], obox_f.at[pl.ds(c * NP + base, RTE)])


_sc_emit = functools.partial(
    pl.kernel,
    out_type=[
        jax.ShapeDtypeStruct((NP,), jnp.float32),
        jax.ShapeDtypeStruct((NP,), jnp.int32),
        jax.ShapeDtypeStruct((4 * NP,), jnp.float32),
    ],
    mesh=_SC_MESH,
    scratch_types=(
        [pltpu.VMEM((RTE,), jnp.float32)] * 8
        + [pltpu.VMEM((RTE,), jnp.int32),
           pltpu.VMEM((RTE,), jnp.float32),
           pltpu.VMEM((RTE,), jnp.int32)]
        + [pltpu.VMEM((RTE,), jnp.float32)] * 4
    ),
)(_emit_body)


@jax.jit
def kernel(x, rois):
    f32 = jnp.float32
    i32 = jnp.int32
    xp = jnp.zeros((NP, 4), f32).at[:N, :].set(x)
    rp = jnp.zeros((NP, 4), f32).at[:N, :].set(rois)
    xj3 = jnp.zeros((NS, 8, RT), f32).at[:, :4, :].set(
        xp.reshape(NS, RT, 4).transpose(0, 2, 1))
    bj3 = jnp.zeros((NS, 8, RT), f32).at[:, :4, :].set(
        rp.reshape(NS, RT, 4).transpose(0, 2, 1))

    at3 = pl.pallas_call(
        _build_body,
        grid=(NS,),
        in_specs=[
            pl.BlockSpec((NP, 4), lambda g: (0, 0)),
            pl.BlockSpec((NP, 4), lambda g: (0, 0)),
            pl.BlockSpec((1, 8, RT), lambda g: (g, 0, 0)),
            pl.BlockSpec((1, 8, RT), lambda g: (g, 0, 0)),
        ],
        out_specs=pl.BlockSpec((1, W, RT), lambda g: (g, 0, 0)),
        out_shape=jax.ShapeDtypeStruct((NS, W, RT), i32),
        scratch_shapes=[pltpu.VMEM((NP, 8), f32)],
    )(xp, rp, xj3, bj3)

    def cond(st):
        return st[1]

    def body(st):
        keep, _ = st
        keep2, diff2 = _sc_solve(at3, keep)
        return keep2[:NP], jnp.any(diff2[:NP] != 0)

    keep0 = jnp.ones((NP,), i32)
    keep_fin, _ = lax.while_loop(cond, body, (keep0, jnp.bool_(True)))

    xt_f = xp.T.reshape(-1)
    rt_f = rp.T.reshape(-1)
    osc, olab, obox_f = _sc_emit(xt_f, rt_f, keep_fin)
    obox = obox_f.reshape(4, NP).T
    return osc[:N], olab[:N], obox[:N, :]


# speedup vs baseline: 1.0806x; 1.0000x over previous
import functools

import jax
import jax.numpy as jnp
from jax import lax
from jax.experimental import pallas as pl
from jax.experimental.pallas import tpu as pltpu
from jax.experimental.pallas import tpu_sc as plsc

N = 5000
NP = 5120
W = 160
NB = 32
NS = 16
RT = NP // NS
L = 16
THR = 0.5
NEG_INF = float("-inf")


def _scores_classes_cols(x4):
    s = jnp.max(x4, axis=1, keepdims=True)
    c0 = x4[:, 0:1] == s
    c1 = x4[:, 1:2] == s
    c2 = x4[:, 2:3] == s
    cls = jnp.where(c0, 0, jnp.where(c1, 1, jnp.where(c2, 2, 3)))
    return s, cls.astype(jnp.int32)


def _build_body(xp_ref, rp_ref, xj_ref, bj_ref, at_ref, scols_ref):
    i32 = jnp.int32
    g = pl.program_id(0)

    @pl.when(g == 0)
    def _prep():
        x4 = xp_ref[:, :]
        s, cls = _scores_classes_cols(x4)
        seff = jnp.where(cls != 3, s, NEG_INF)
        r4 = rp_ref[:, :]
        area = (jnp.maximum(r4[:, 2:3] - r4[:, 0:1], 0.0)
                * jnp.maximum(r4[:, 3:4] - r4[:, 1:2], 0.0))
        scols_ref[:, :] = jnp.concatenate(
            [r4, area, seff, seff, seff], axis=1)

    xjb = xj_ref[0]
    sj = jnp.max(xjb[0:4, :], axis=0, keepdims=True)
    inv_j = ((xjb[3:4, :] > xjb[0:1, :])
             & (xjb[3:4, :] > xjb[1:2, :])
             & (xjb[3:4, :] > xjb[2:3, :]))
    sj_eff = jnp.where(inv_j, NEG_INF, sj)
    bjb = bj_ref[0]
    x1j, y1j = bjb[0:1, :], bjb[1:2, :]
    x2j, y2j = bjb[2:3, :], bjb[3:4, :]
    area_j = jnp.maximum(x2j - x1j, 0.0) * jnp.maximum(y2j - y1j, 0.0)
    idx_j = lax.broadcasted_iota(i32, (1, RT), 1) + g * RT

    acc = jnp.zeros((W, RT), i32)
    for b in range(NB):
        sc = scols_ref[W * b:W * (b + 1), :]
        x1i, y1i = sc[:, 0:1], sc[:, 1:2]
        x2i, y2i = sc[:, 2:3], sc[:, 3:4]
        area_i = sc[:, 4:5]
        si_eff = sc[:, 5:6]
        idx_i = lax.broadcasted_iota(i32, (W, 1), 0) + W * b

        iw = jnp.maximum(jnp.minimum(x2j, x2i) - jnp.maximum(x1j, x1i), 0.0)
        ih = jnp.maximum(jnp.minimum(y2j, y2i) - jnp.maximum(y1j, y1i), 0.0)
        inter = iw * ih
        union = jnp.maximum(area_i + area_j - inter, 1e-9)
        conflict = inter > THR * union
        prec = (si_eff > sj_eff) | ((si_eff == sj_eff) & (idx_i < idx_j))
        acc = acc | lax.shift_left((conflict & prec).astype(i32), b)
    at_ref[0] = acc


NR = 12


def _solve_body(at3, keep_in, keep_out, diff_out,
                at_v, keep_v, nk_v, kw_v, ko_v, shk):
    i32 = jnp.int32
    cid = lax.axis_index("c")
    sid = lax.axis_index("s")
    tid = sid
    base = pl.multiple_of(tid * RT, RT)
    pltpu.sync_copy(at3.at[tid], at_v)
    pltpu.sync_copy(keep_in, keep_v)

    nch = RT // L

    def one_round(src, dst, slot):
        diff = jnp.zeros((L,), i32)
        for c in range(W // L):
            kwc = jnp.zeros((L,), i32)
            for b in range(NB):
                kwc = kwc | lax.shift_left(
                    src[pl.ds(W * b + L * c, L)], b)
            diff = diff | (kwc ^ kw_v[pl.ds(L * c, L)])
            kw_v[pl.ds(L * c, L)] = kwc

        @plsc.parallel_loop(
            0, W // L, unroll=2,
            carry=tuple(jnp.zeros((L,), i32) for _ in range(nch)))
        def accs(wc, accs):
            woff = pl.multiple_of(L * wc, L)
            kwc = kw_v[pl.ds(woff, L)]
            for l in range(L):
                kb = jnp.full((L,), kwc[l], i32)
                w = woff + l
                accs = tuple(
                    accs[rc] | (at_v[w, pl.ds(L * rc, L)] & kb)
                    for rc in range(nch))
            return accs
        for rc in range(nch):
            ko_v[pl.ds(L * rc, L)] = jnp.where(
                accs[rc] == 0, 1, 0).astype(i32)

        pltpu.sync_copy(ko_v, shk.at[pl.ds(slot * NP + base, RT)])
        plsc.subcore_barrier()
        pltpu.sync_copy(shk.at[pl.ds(slot * NP, NP)], dst)
        return diff

    def round_pair(r, _):
        one_round(keep_v, nk_v, 0)
        return one_round(nk_v, keep_v, 1)

    zl = jnp.zeros((L,), i32)
    for c in range(W // L):
        kw_v[pl.ds(L * c, L)] = zl
    lax.fori_loop(0, NR // 2, round_pair, zl)
    diff = zl
    for c in range(W // L):
        kwc = jnp.zeros((L,), i32)
        for b in range(NB):
            kwc = kwc | lax.shift_left(
                keep_v[pl.ds(W * b + L * c, L)], b)
        diff = diff | (kwc ^ kw_v[pl.ds(L * c, L)])

    obase = pl.multiple_of(cid * NP, NP) + base
    for rc in range(nch):
        ko_v[pl.ds(L * rc, L)] = keep_v[pl.ds(base + L * rc, L)]
    pltpu.sync_copy(ko_v, keep_out.at[pl.ds(obase, RT)])
    for rc in range(nch):
        ko_v[pl.ds(L * rc, L)] = diff
    pltpu.sync_copy(ko_v, diff_out.at[pl.ds(obase, RT)])


_SC_MESH = plsc.VectorSubcoreMesh(core_axis_name="c", subcore_axis_name="s")

_sc_solve = functools.partial(
    pl.kernel,
    out_type=[
        jax.ShapeDtypeStruct((2 * NP,), jnp.int32),
        jax.ShapeDtypeStruct((2 * NP,), jnp.int32),
    ],
    mesh=_SC_MESH,
    scratch_types=[
        pltpu.VMEM((W, RT), jnp.int32),
        pltpu.VMEM((NP,), jnp.int32),
        pltpu.VMEM((NP,), jnp.int32),
        pltpu.VMEM((W,), jnp.int32),
        pltpu.VMEM((RT,), jnp.int32),
        pltpu.VMEM_SHARED((2 * NP,), jnp.int32),
    ],
)(_solve_body)


RTE = NP // 32


def _emit_body(xt_f, rt_f, keep_in, osc, olab, obox_f,
               xc0, xc1, xc2, xc3, rc0, rc1, rc2, rc3,
               keep_v, osc_v, olab_v, ob0, ob1, ob2, ob3):
    f32 = jnp.float32
    i32 = jnp.int32
    wid = lax.axis_index("s") * 2 + lax.axis_index("c")
    base = pl.multiple_of(wid * RTE, RTE)

    xcs = (xc0, xc1, xc2, xc3)
    rcs = (rc0, rc1, rc2, rc3)
    obs = (ob0, ob1, ob2, ob3)
    for c in range(4):
        pltpu.sync_copy(xt_f.at[pl.ds(c * NP + base, RTE)], xcs[c])
        pltpu.sync_copy(rt_f.at[pl.ds(c * NP + base, RTE)], rcs[c])
    pltpu.sync_copy(keep_in.at[pl.ds(base, RTE)], keep_v)

    for k in range(RTE // L):
        sl = pl.ds(L * k, L)
        x0, x1, x2, x3 = xc0[sl], xc1[sl], xc2[sl], xc3[sl]
        s = jnp.maximum(jnp.maximum(x0, x1), jnp.maximum(x2, x3))
        is3 = (x3 > x0) & (x3 > x1) & (x3 > x2)
        cls = jnp.where(x0 == s, 0,
                        jnp.where(x1 == s, 1,
                                  jnp.where(x2 == s, 2, 3))).astype(i32)
        kv = keep_v[sl] * jnp.where(is3, 0, 1)
        kf = kv.astype(f32)
        osc_v[sl] = s * kf
        olab_v[sl] = cls * kv
        for c in range(4):
            obs[c][sl] = rcs[c][sl] * kf
    pltpu.sync_copy(osc_v, osc.at[pl.ds(base, RTE)])
    pltpu.sync_copy(olab_v, olab.at[pl.ds(base, RTE)])
    for c in range(4):
        pltpu.sync_copy(obs[c], obox_f.at[pl.ds(c * NP + base, RTE)])


_sc_emit = functools.partial(
    pl.kernel,
    out_type=[
        jax.ShapeDtypeStruct((NP,), jnp.float32),
        jax.ShapeDtypeStruct((NP,), jnp.int32),
        jax.ShapeDtypeStruct((4 * NP,), jnp.float32),
    ],
    mesh=_SC_MESH,
    scratch_types=(
        [pltpu.VMEM((RTE,), jnp.float32)] * 8
        + [pltpu.VMEM((RTE,), jnp.int32),
           pltpu.VMEM((RTE,), jnp.float32),
           pltpu.VMEM((RTE,), jnp.int32)]
        + [pltpu.VMEM((RTE,), jnp.float32)] * 4
    ),
)(_emit_body)


@jax.jit
def kernel(x, rois):
    f32 = jnp.float32
    i32 = jnp.int32
    xp = jnp.zeros((NP, 4), f32).at[:N, :].set(x)
    rp = jnp.zeros((NP, 4), f32).at[:N, :].set(rois)
    xj3 = jnp.zeros((NS, 8, RT), f32).at[:, :4, :].set(
        xp.reshape(NS, RT, 4).transpose(0, 2, 1))
    bj3 = jnp.zeros((NS, 8, RT), f32).at[:, :4, :].set(
        rp.reshape(NS, RT, 4).transpose(0, 2, 1))

    at3 = pl.pallas_call(
        _build_body,
        grid=(NS,),
        in_specs=[
            pl.BlockSpec((NP, 4), lambda g: (0, 0)),
            pl.BlockSpec((NP, 4), lambda g: (0, 0)),
            pl.BlockSpec((1, 8, RT), lambda g: (g, 0, 0)),
            pl.BlockSpec((1, 8, RT), lambda g: (g, 0, 0)),
        ],
        out_specs=pl.BlockSpec((1, W, RT), lambda g: (g, 0, 0)),
        out_shape=jax.ShapeDtypeStruct((NS, W, RT), i32),
        scratch_shapes=[pltpu.VMEM((NP, 8), f32)],
    )(xp, rp, xj3, bj3)

    def cond(st):
        return st[1]

    def body(st):
        keep, _ = st
        keep2, diff2 = _sc_solve(at3, keep)
        return keep2[:NP], jnp.any(diff2[:NP] != 0)

    keep0 = jnp.ones((NP,), i32)
    keep_fin, _ = lax.while_loop(cond, body, (keep0, jnp.bool_(True)))

    xt_f = xp.T.reshape(-1)
    rt_f = rp.T.reshape(-1)
    osc, olab, obox_f = _sc_emit(xt_f, rt_f, keep_fin)
    obox = obox_f.reshape(4, NP).T
    return osc[:N], olab[:N], obox[:N, :]


# speedup vs baseline: 1.0815x; 1.0008x over previous
import functools

import jax
import jax.numpy as jnp
from jax import lax
from jax.experimental import pallas as pl
from jax.experimental.pallas import tpu as pltpu
from jax.experimental.pallas import tpu_sc as plsc

N = 5000
NP = 5120
W = 160
NB = 32
NS = 16
RT = NP // NS
L = 16
THR = 0.5
NEG_INF = float("-inf")


def _scores_classes_cols(x4):
    s = jnp.max(x4, axis=1, keepdims=True)
    c0 = x4[:, 0:1] == s
    c1 = x4[:, 1:2] == s
    c2 = x4[:, 2:3] == s
    cls = jnp.where(c0, 0, jnp.where(c1, 1, jnp.where(c2, 2, 3)))
    return s, cls.astype(jnp.int32)


def _build_body(xp_ref, rp_ref, xj_ref, bj_ref, at_ref, scols_ref):
    i32 = jnp.int32
    g = pl.program_id(0)

    @pl.when(g == 0)
    def _prep():
        x4 = xp_ref[:, :]
        s, cls = _scores_classes_cols(x4)
        seff = jnp.where(cls != 3, s, NEG_INF)
        r4 = rp_ref[:, :]
        area = (jnp.maximum(r4[:, 2:3] - r4[:, 0:1], 0.0)
                * jnp.maximum(r4[:, 3:4] - r4[:, 1:2], 0.0))
        scols_ref[:, :] = jnp.concatenate(
            [r4, area, seff, seff, seff], axis=1)

    xjb = xj_ref[0]
    sj = jnp.max(xjb[0:4, :], axis=0, keepdims=True)
    inv_j = ((xjb[3:4, :] > xjb[0:1, :])
             & (xjb[3:4, :] > xjb[1:2, :])
             & (xjb[3:4, :] > xjb[2:3, :]))
    sj_eff = jnp.where(inv_j, NEG_INF, sj)
    bjb = bj_ref[0]
    x1j, y1j = bjb[0:1, :], bjb[1:2, :]
    x2j, y2j = bjb[2:3, :], bjb[3:4, :]
    area_j = jnp.maximum(x2j - x1j, 0.0) * jnp.maximum(y2j - y1j, 0.0)
    idx_j = lax.broadcasted_iota(i32, (1, RT), 1) + g * RT

    acc = jnp.zeros((W, RT), i32)
    for b in range(NB):
        sc = scols_ref[W * b:W * (b + 1), :]
        x1i, y1i = sc[:, 0:1], sc[:, 1:2]
        x2i, y2i = sc[:, 2:3], sc[:, 3:4]
        area_i = sc[:, 4:5]
        si_eff = sc[:, 5:6]
        idx_i = lax.broadcasted_iota(i32, (W, 1), 0) + W * b

        iw = jnp.maximum(jnp.minimum(x2j, x2i) - jnp.maximum(x1j, x1i), 0.0)
        ih = jnp.maximum(jnp.minimum(y2j, y2i) - jnp.maximum(y1j, y1i), 0.0)
        inter = iw * ih
        union = jnp.maximum(area_i + area_j - inter, 1e-9)
        conflict = inter > THR * union
        prec = (si_eff > sj_eff) | ((si_eff == sj_eff) & (idx_i < idx_j))
        acc = acc | lax.shift_left((conflict & prec).astype(i32), b)
    at_ref[0] = acc


NR = 12


def _solve_body(at3, keep_in, keep_out, diff_out,
                at_v, keep_v, nk_v, kw_v, ko_v, shk):
    i32 = jnp.int32
    cid = lax.axis_index("c")
    sid = lax.axis_index("s")
    tid = sid
    base = pl.multiple_of(tid * RT, RT)
    pltpu.sync_copy(at3.at[tid], at_v)
    pltpu.sync_copy(keep_in, keep_v)

    nch = RT // L

    def one_round(src, dst, slot):
        diff = jnp.zeros((L,), i32)
        for c in range(W // L):
            kwc = jnp.zeros((L,), i32)
            for b in range(NB):
                kwc = kwc | lax.shift_left(
                    src[pl.ds(W * b + L * c, L)], b)
            diff = diff | (kwc ^ kw_v[pl.ds(L * c, L)])
            kw_v[pl.ds(L * c, L)] = kwc

        def wcbody(wc, accs):
            woff = pl.multiple_of(L * wc, L)
            kwc = kw_v[pl.ds(woff, L)]
            for l in range(L):
                kb = jnp.full((L,), kwc[l], i32)
                w = woff + l
                accs = tuple(
                    accs[rc] | (at_v[w, pl.ds(L * rc, L)] & kb)
                    for rc in range(nch))
            return accs

        accs = lax.fori_loop(
            0, W // L, wcbody,
            tuple(jnp.zeros((L,), i32) for _ in range(nch)))
        for rc in range(nch):
            ko_v[pl.ds(L * rc, L)] = jnp.where(
                accs[rc] == 0, 1, 0).astype(i32)

        pltpu.sync_copy(ko_v, shk.at[pl.ds(slot * NP + base, RT)])
        plsc.subcore_barrier()
        pltpu.sync_copy(shk.at[pl.ds(slot * NP, NP)], dst)
        return diff

    def round_pair(r, _):
        one_round(keep_v, nk_v, 0)
        return one_round(nk_v, keep_v, 1)

    zl = jnp.zeros((L,), i32)
    for c in range(W // L):
        kw_v[pl.ds(L * c, L)] = zl
    lax.fori_loop(0, NR // 2, round_pair, zl)
    diff = zl
    for c in range(W // L):
        kwc = jnp.zeros((L,), i32)
        for b in range(NB):
            kwc = kwc | lax.shift_left(
                keep_v[pl.ds(W * b + L * c, L)], b)
        diff = diff | (kwc ^ kw_v[pl.ds(L * c, L)])

    obase = pl.multiple_of(cid * NP, NP) + base
    for rc in range(nch):
        ko_v[pl.ds(L * rc, L)] = keep_v[pl.ds(base + L * rc, L)]
    pltpu.sync_copy(ko_v, keep_out.at[pl.ds(obase, RT)])
    for rc in range(nch):
        ko_v[pl.ds(L * rc, L)] = diff
    pltpu.sync_copy(ko_v, diff_out.at[pl.ds(obase, RT)])


_SC_MESH = plsc.VectorSubcoreMesh(core_axis_name="c", subcore_axis_name="s")

_sc_solve = functools.partial(
    pl.kernel,
    out_type=[
        jax.ShapeDtypeStruct((2 * NP,), jnp.int32),
        jax.ShapeDtypeStruct((2 * NP,), jnp.int32),
    ],
    mesh=_SC_MESH,
    scratch_types=[
        pltpu.VMEM((W, RT), jnp.int32),
        pltpu.VMEM((NP,), jnp.int32),
        pltpu.VMEM((NP,), jnp.int32),
        pltpu.VMEM((W,), jnp.int32),
        pltpu.VMEM((RT,), jnp.int32),
        pltpu.VMEM_SHARED((2 * NP,), jnp.int32),
    ],
)(_solve_body)


RTE = NP // 32


def _emit_body(xt_f, rt_f, keep_in, osc, olab, obox_f,
               xc0, xc1, xc2, xc3, rc0, rc1, rc2, rc3,
               keep_v, osc_v, olab_v, ob0, ob1, ob2, ob3):
    f32 = jnp.float32
    i32 = jnp.int32
    wid = lax.axis_index("s") * 2 + lax.axis_index("c")
    base = pl.multiple_of(wid * RTE, RTE)

    xcs = (xc0, xc1, xc2, xc3)
    rcs = (rc0, rc1, rc2, rc3)
    obs = (ob0, ob1, ob2, ob3)
    for c in range(4):
        pltpu.sync_copy(xt_f.at[pl.ds(c * NP + base, RTE)], xcs[c])
        pltpu.sync_copy(rt_f.at[pl.ds(c * NP + base, RTE)], rcs[c])
    pltpu.sync_copy(keep_in.at[pl.ds(base, RTE)], keep_v)

    for k in range(RTE // L):
        sl = pl.ds(L * k, L)
        x0, x1, x2, x3 = xc0[sl], xc1[sl], xc2[sl], xc3[sl]
        s = jnp.maximum(jnp.maximum(x0, x1), jnp.maximum(x2, x3))
        is3 = (x3 > x0) & (x3 > x1) & (x3 > x2)
        cls = jnp.where(x0 == s, 0,
                        jnp.where(x1 == s, 1,
                                  jnp.where(x2 == s, 2, 3))).astype(i32)
        kv = keep_v[sl] * jnp.where(is3, 0, 1)
        kf = kv.astype(f32)
        osc_v[sl] = s * kf
        olab_v[sl] = cls * kv
        for c in range(4):
            obs[c][sl] = rcs[c][sl] * kf
    pltpu.sync_copy(osc_v, osc.at[pl.ds(base, RTE)])
    pltpu.sync_copy(olab_v, olab.at[pl.ds(base, RTE)])
    for c in range(4):
        pltpu.sync_copy(obs[c], obox_f.at[pl.ds(c * NP + base, RTE)])


_sc_emit = functools.partial(
    pl.kernel,
    out_type=[
        jax.ShapeDtypeStruct((NP,), jnp.float32),
        jax.ShapeDtypeStruct((NP,), jnp.int32),
        jax.ShapeDtypeStruct((4 * NP,), jnp.float32),
    ],
    mesh=_SC_MESH,
    scratch_types=(
        [pltpu.VMEM((RTE,), jnp.float32)] * 8
        + [pltpu.VMEM((RTE,), jnp.int32),
           pltpu.VMEM((RTE,), jnp.float32),
           pltpu.VMEM((RTE,), jnp.int32)]
        + [pltpu.VMEM((RTE,), jnp.float32)] * 4
    ),
)(_emit_body)


@jax.jit
def kernel(x, rois):
    f32 = jnp.float32
    i32 = jnp.int32
    xp = jnp.zeros((NP, 4), f32).at[:N, :].set(x)
    rp = jnp.zeros((NP, 4), f32).at[:N, :].set(rois)
    xj3 = jnp.zeros((NS, 8, RT), f32).at[:, :4, :].set(
        xp.reshape(NS, RT, 4).transpose(0, 2, 1))
    bj3 = jnp.zeros((NS, 8, RT), f32).at[:, :4, :].set(
        rp.reshape(NS, RT, 4).transpose(0, 2, 1))

    at3 = pl.pallas_call(
        _build_body,
        grid=(NS,),
        in_specs=[
            pl.BlockSpec((NP, 4), lambda g: (0, 0)),
            pl.BlockSpec((NP, 4), lambda g: (0, 0)),
            pl.BlockSpec((1, 8, RT), lambda g: (g, 0, 0)),
            pl.BlockSpec((1, 8, RT), lambda g: (g, 0, 0)),
        ],
        out_specs=pl.BlockSpec((1, W, RT), lambda g: (g, 0, 0)),
        out_shape=jax.ShapeDtypeStruct((NS, W, RT), i32),
        scratch_shapes=[pltpu.VMEM((NP, 8), f32)],
    )(xp, rp, xj3, bj3)

    def cond(st):
        return st[1]

    def body(st):
        keep, _ = st
        keep2, diff2 = _sc_solve(at3, keep)
        return keep2[:NP], jnp.any(diff2[:NP] != 0)

    keep0 = jnp.ones((NP,), i32)
    keep_fin, _ = lax.while_loop(cond, body, (keep0, jnp.bool_(True)))

    xt_f = xp.T.reshape(-1)
    rt_f = rp.T.reshape(-1)
    osc, olab, obox_f = _sc_emit(xt_f, rt_f, keep_fin)
    obox = obox_f.reshape(4, NP).T
    return osc[:N], olab[:N], obox[:N, :]
